# Initial kernel scaffold; baseline (speedup 1.0000x reference)
import jax
import jax.numpy as jnp
from jax.experimental import pallas as pl


def kernel(atom_in_fea, nbr_fea, nbr_fea_idx, W_full, b_full, bn1_gamma, bn1_beta, bn2_gamma, bn2_beta):
    raise NotImplementedError("write your pallas kernel here")


# speedup vs baseline: 1.8124x; 1.8124x over previous
import functools
import jax
import jax.numpy as jnp
from jax import lax
from jax.experimental import pallas as pl
from jax.experimental.pallas import tpu as pltpu
from jax.experimental.pallas import tpu_sc as plsc

A = 128
NBR = 16
EPS = 1e-5


def _proj_body(atom_ref, ws_ref, wn_ref, b_ref, p_ref, t_ref):
    x = atom_ref[...]
    p_ref[...] = jnp.dot(x, ws_ref[...], preferred_element_type=jnp.float32) + b_ref[...]
    t_ref[...] = jnp.dot(x, wn_ref[...], preferred_element_type=jnp.float32)


def _projections(atom, ws, wn, b2d, tile):
    n = atom.shape[0]
    grid = (n // tile,)
    return pl.pallas_call(
        _proj_body,
        grid=grid,
        in_specs=[
            pl.BlockSpec((tile, A), lambda i: (i, 0)),
            pl.BlockSpec((A, 2 * A), lambda i: (0, 0)),
            pl.BlockSpec((A, 2 * A), lambda i: (0, 0)),
            pl.BlockSpec((1, 2 * A), lambda i: (0, 0)),
        ],
        out_specs=[
            pl.BlockSpec((tile, 2 * A), lambda i: (i, 0)),
            pl.BlockSpec((tile, 2 * A), lambda i: (i, 0)),
        ],
        out_shape=[
            jax.ShapeDtypeStruct((n, 2 * A), jnp.float32),
            jax.ShapeDtypeStruct((n, 2 * A), jnp.float32),
        ],
    )(atom, ws, wn, b2d)


def _make_sc_gather(nrows, d, chunk):
    info = plsc.get_sparse_core_info()
    nw = info.num_cores * info.num_subcores
    per_w = nrows // nw
    nch = per_w // chunk
    mesh = plsc.VectorSubcoreMesh(core_axis_name="c", subcore_axis_name="s")

    @functools.partial(
        pl.kernel,
        mesh=mesh,
        out_type=jax.ShapeDtypeStruct((nrows, d), jnp.float32),
        scratch_types=[
            pltpu.VMEM((chunk,), jnp.int32),
            pltpu.VMEM((chunk, d), jnp.float32),
            pltpu.SemaphoreType.DMA,
        ],
    )
    def gk(t_hbm, idx_hbm, out_hbm, idx_v, rows_v, sem):
        wid = lax.axis_index("s") * info.num_cores + lax.axis_index("c")
        base = wid * per_w

        def body(ci, carry):
            off = base + ci * chunk
            pltpu.sync_copy(idx_hbm.at[pl.ds(off, chunk)], idx_v)
            pltpu.async_copy(t_hbm.at[idx_v], rows_v, sem).wait()
            pltpu.sync_copy(rows_v, out_hbm.at[pl.ds(off, chunk)])
            return carry

        lax.fori_loop(0, nch, body, 0)

    return gk


def _stats_body(g_ref, nbr_ref, p_ref, we_ref, sum_ref, sq_ref):
    j = pl.program_id(0)
    n = pl.program_id(1)
    v = g_ref[...] + p_ref[...] + jnp.dot(
        nbr_ref[...], we_ref[...], preferred_element_type=jnp.float32)

    @pl.when(jnp.logical_and(j == 0, n == 0))
    def _():
        sum_ref[...] = jnp.zeros_like(sum_ref)
        sq_ref[...] = jnp.zeros_like(sq_ref)

    sum_ref[...] += jnp.sum(v, axis=0, keepdims=True)
    sq_ref[...] += jnp.sum(v * v, axis=0, keepdims=True)


def _stats(g, nbr_t, p, we, m, n, tile):
    nb = n // tile
    return pl.pallas_call(
        _stats_body,
        grid=(m, nb),
        in_specs=[
            pl.BlockSpec((tile, 2 * A), lambda j, i: (j * nb + i, 0)),
            pl.BlockSpec((tile, NBR), lambda j, i: (j * nb + i, 0)),
            pl.BlockSpec((tile, 2 * A), lambda j, i: (i, 0)),
            pl.BlockSpec((NBR, 2 * A), lambda j, i: (0, 0)),
        ],
        out_specs=[
            pl.BlockSpec((1, 2 * A), lambda j, i: (0, 0)),
            pl.BlockSpec((1, 2 * A), lambda j, i: (0, 0)),
        ],
        out_shape=[
            jax.ShapeDtypeStruct((1, 2 * A), jnp.float32),
            jax.ShapeDtypeStruct((1, 2 * A), jnp.float32),
        ],
    )(g, nbr_t, p, we)


def _apply_body(g_ref, nbr_ref, p_ref, we_ref, sc_ref, sh_ref,
                s_ref, ssum_ref, ssq_ref, *, m):
    n = pl.program_id(0)
    j = pl.program_id(1)
    v = g_ref[...] + p_ref[...] + jnp.dot(
        nbr_ref[...], we_ref[...], preferred_element_type=jnp.float32)
    u = v * sc_ref[...] + sh_ref[...]
    f = jax.nn.sigmoid(u[:, :A])
    c = jnp.maximum(u[:, A:], 0.0)
    prod = f * c

    @pl.when(j == 0)
    def _():
        s_ref[...] = prod

    @pl.when(j > 0)
    def _():
        s_ref[...] += prod

    @pl.when(jnp.logical_and(n == 0, j == m - 1))
    def _():
        ssum_ref[...] = jnp.zeros_like(ssum_ref)
        ssq_ref[...] = jnp.zeros_like(ssq_ref)

    @pl.when(j == m - 1)
    def _():
        s = s_ref[...]
        ssum_ref[...] += jnp.sum(s, axis=0, keepdims=True)
        ssq_ref[...] += jnp.sum(s * s, axis=0, keepdims=True)


def _apply(g, nbr_t, p, we, scale, shift, m, n, tile):
    nb = n // tile
    return pl.pallas_call(
        functools.partial(_apply_body, m=m),
        grid=(nb, m),
        in_specs=[
            pl.BlockSpec((tile, 2 * A), lambda i, j: (j * nb + i, 0)),
            pl.BlockSpec((tile, NBR), lambda i, j: (j * nb + i, 0)),
            pl.BlockSpec((tile, 2 * A), lambda i, j: (i, 0)),
            pl.BlockSpec((NBR, 2 * A), lambda i, j: (0, 0)),
            pl.BlockSpec((1, 2 * A), lambda i, j: (0, 0)),
            pl.BlockSpec((1, 2 * A), lambda i, j: (0, 0)),
        ],
        out_specs=[
            pl.BlockSpec((tile, A), lambda i, j: (i, 0)),
            pl.BlockSpec((1, A), lambda i, j: (0, 0)),
            pl.BlockSpec((1, A), lambda i, j: (0, 0)),
        ],
        out_shape=[
            jax.ShapeDtypeStruct((n, A), jnp.float32),
            jax.ShapeDtypeStruct((1, A), jnp.float32),
            jax.ShapeDtypeStruct((1, A), jnp.float32),
        ],
    )(g, nbr_t, p, we, scale, shift)


def _final_body(atom_ref, s_ref, sc_ref, sh_ref, out_ref):
    out_ref[...] = jnp.maximum(
        atom_ref[...] + s_ref[...] * sc_ref[...] + sh_ref[...], 0.0)


def _final(atom, s, scale2, shift2, tile):
    n = atom.shape[0]
    return pl.pallas_call(
        _final_body,
        grid=(n // tile,),
        in_specs=[
            pl.BlockSpec((tile, A), lambda i: (i, 0)),
            pl.BlockSpec((tile, A), lambda i: (i, 0)),
            pl.BlockSpec((1, A), lambda i: (0, 0)),
            pl.BlockSpec((1, A), lambda i: (0, 0)),
        ],
        out_specs=pl.BlockSpec((tile, A), lambda i: (i, 0)),
        out_shape=jax.ShapeDtypeStruct((n, A), jnp.float32),
    )(atom, s, scale2, shift2)


def kernel(atom_in_fea, nbr_fea, nbr_fea_idx, W_full, b_full,
           bn1_gamma, bn1_beta, bn2_gamma, bn2_beta):
    n, m = nbr_fea_idx.shape
    ws = W_full[:A]
    wn = W_full[A:2 * A]
    we = W_full[2 * A:]
    b2d = b_full.reshape(1, 2 * A)

    p, t = _projections(atom_in_fea, ws, wn, b2d, tile=2000)

    idx_t = nbr_fea_idx.astype(jnp.int32).T.reshape(-1)
    nbr_t = nbr_fea.transpose(1, 0, 2).reshape(m * n, NBR)

    g = _make_sc_gather(m * n, 2 * A, chunk=200)(t, idx_t)

    vsum, vsq = _stats(g, nbr_t, p, we, m, n, tile=2000)
    cnt = float(n * m)
    mu = vsum / cnt
    var = vsq / cnt - mu * mu
    scale = (bn1_gamma / jnp.sqrt(var + EPS)).reshape(1, 2 * A)
    shift = (bn1_beta - mu * scale).reshape(1, 2 * A)

    s, ssum, ssq = _apply(g, nbr_t, p, we, scale, shift, m, n, tile=2000)
    mu2 = ssum / float(n)
    var2 = ssq / float(n) - mu2 * mu2
    scale2 = (bn2_gamma / jnp.sqrt(var2 + EPS)).reshape(1, A)
    shift2 = (bn2_beta - mu2 * scale2).reshape(1, A)

    return _final(atom_in_fea, s, scale2, shift2, tile=2000)


# speedup vs baseline: 2.4010x; 1.3248x over previous
import functools
import jax
import jax.numpy as jnp
from jax import lax
from jax.experimental import pallas as pl
from jax.experimental.pallas import tpu as pltpu
from jax.experimental.pallas import tpu_sc as plsc

A = 128
NBR = 16
EPS = 1e-5


_MASK_HI = -65536


def _unpack(word):
    lo = lax.bitcast_convert_type(jnp.left_shift(word, 16), jnp.float32)
    hi = lax.bitcast_convert_type(jnp.bitwise_and(word, _MASK_HI), jnp.float32)
    return lo, hi


def _proj_body(atom_ref, ws_ref, wn_ref, b_ref, p_ref, t_ref):
    x = atom_ref[...]
    p_ref[...] = jnp.dot(x, ws_ref[...], preferred_element_type=jnp.float32) + b_ref[...]
    t = jnp.dot(x, wn_ref[...], preferred_element_type=jnp.float32)
    lo = lax.bitcast_convert_type(t[:, :A], jnp.int32) + 0x8000
    hi = lax.bitcast_convert_type(t[:, A:], jnp.int32) + 0x8000
    t_ref[...] = jnp.bitwise_or(
        jnp.bitwise_and(hi, _MASK_HI), lax.shift_right_logical(lo, 16))


def _projections(atom, ws, wn, b2d, tile):
    n = atom.shape[0]
    grid = (n // tile,)
    return pl.pallas_call(
        _proj_body,
        grid=grid,
        in_specs=[
            pl.BlockSpec((tile, A), lambda i: (i, 0)),
            pl.BlockSpec((A, 2 * A), lambda i: (0, 0)),
            pl.BlockSpec((A, 2 * A), lambda i: (0, 0)),
            pl.BlockSpec((1, 2 * A), lambda i: (0, 0)),
        ],
        out_specs=[
            pl.BlockSpec((tile, 2 * A), lambda i: (i, 0)),
            pl.BlockSpec((tile, A), lambda i: (i, 0)),
        ],
        out_shape=[
            jax.ShapeDtypeStruct((n, 2 * A), jnp.float32),
            jax.ShapeDtypeStruct((n, A), jnp.int32),
        ],
    )(atom, ws, wn, b2d)


def _make_sc_gather(nrows, d, chunk, dtype):
    info = plsc.get_sparse_core_info()
    nw = info.num_cores * info.num_subcores
    per_w = nrows // nw
    nch = per_w // chunk
    mesh = plsc.VectorSubcoreMesh(core_axis_name="c", subcore_axis_name="s")

    @functools.partial(
        pl.kernel,
        mesh=mesh,
        out_type=jax.ShapeDtypeStruct((nrows, d), dtype),
        scratch_types=[
            pltpu.VMEM((per_w,), jnp.int32),
            pltpu.VMEM((chunk, d), dtype),
            pltpu.VMEM((chunk, d), dtype),
            pltpu.SemaphoreType.DMA,
            pltpu.SemaphoreType.DMA,
            pltpu.SemaphoreType.DMA,
            pltpu.SemaphoreType.DMA,
        ],
    )
    def gk(t_hbm, idx_hbm, out_hbm, idx_all, r0, r1, sg0, sg1, sw0, sw1):
        wid = lax.axis_index("s") * info.num_cores + lax.axis_index("c")
        base = wid * per_w
        pltpu.sync_copy(idx_hbm.at[pl.ds(base, per_w)], idx_all)
        bufs = (r0, r1)
        sgs = (sg0, sg1)
        sws = (sw0, sw1)

        def start_gather(k, b):
            pltpu.async_copy(
                t_hbm.at[idx_all.at[pl.ds(k * chunk, chunk)]], bufs[b], sgs[b])

        for b in range(2):
            start_gather(b, b)

        def body(o, carry):
            for b in range(2):
                k = o * 2 + b
                pltpu.make_async_copy(
                    t_hbm.at[idx_all.at[pl.ds(k * chunk, chunk)]],
                    bufs[b], sgs[b]).wait()
                dst = out_hbm.at[pl.ds(base + k * chunk, chunk)]
                pltpu.async_copy(bufs[b], dst, sws[b])
                pltpu.make_async_copy(bufs[b], dst, sws[b]).wait()

                @pl.when(k + 2 < nch)
                def _():
                    start_gather(k + 2, b)
            return carry

        lax.fori_loop(0, nch // 2, body, 0)

    return gk


def _stats_body(g_ref, nbr_ref, p_ref, we_ref, sum_ref, sq_ref):
    j = pl.program_id(0)
    n = pl.program_id(1)
    tf, tc = _unpack(g_ref[...])
    ep = p_ref[...] + jnp.dot(
        nbr_ref[...], we_ref[...], preferred_element_type=jnp.float32)
    v = jnp.concatenate([tf + ep[:, :A], tc + ep[:, A:]], axis=1)

    @pl.when(jnp.logical_and(j == 0, n == 0))
    def _():
        sum_ref[...] = jnp.zeros_like(sum_ref)
        sq_ref[...] = jnp.zeros_like(sq_ref)

    sum_ref[...] += jnp.sum(v, axis=0, keepdims=True)
    sq_ref[...] += jnp.sum(v * v, axis=0, keepdims=True)


def _stats(g, nbr_t, p, we, m, n, tile):
    nb = n // tile
    return pl.pallas_call(
        _stats_body,
        grid=(m, nb),
        in_specs=[
            pl.BlockSpec((tile, A), lambda j, i: (j * nb + i, 0)),
            pl.BlockSpec((tile, NBR), lambda j, i: (j * nb + i, 0)),
            pl.BlockSpec((tile, 2 * A), lambda j, i: (i, 0)),
            pl.BlockSpec((NBR, 2 * A), lambda j, i: (0, 0)),
        ],
        out_specs=[
            pl.BlockSpec((1, 2 * A), lambda j, i: (0, 0)),
            pl.BlockSpec((1, 2 * A), lambda j, i: (0, 0)),
        ],
        out_shape=[
            jax.ShapeDtypeStruct((1, 2 * A), jnp.float32),
            jax.ShapeDtypeStruct((1, 2 * A), jnp.float32),
        ],
    )(g, nbr_t, p, we)


def _apply_body(g_ref, nbr_ref, p_ref, we_ref, sc_ref, sh_ref,
                s_ref, ssum_ref, ssq_ref, *, m):
    n = pl.program_id(0)
    j = pl.program_id(1)
    tf, tc = _unpack(g_ref[...])
    ep = p_ref[...] + jnp.dot(
        nbr_ref[...], we_ref[...], preferred_element_type=jnp.float32)
    sc = sc_ref[...]
    sh = sh_ref[...]
    uf = (tf + ep[:, :A]) * sc[:, :A] + sh[:, :A]
    uc = (tc + ep[:, A:]) * sc[:, A:] + sh[:, A:]
    prod = jax.nn.sigmoid(uf) * jnp.maximum(uc, 0.0)

    @pl.when(j == 0)
    def _():
        s_ref[...] = prod

    @pl.when(j > 0)
    def _():
        s_ref[...] += prod

    @pl.when(jnp.logical_and(n == 0, j == m - 1))
    def _():
        ssum_ref[...] = jnp.zeros_like(ssum_ref)
        ssq_ref[...] = jnp.zeros_like(ssq_ref)

    @pl.when(j == m - 1)
    def _():
        s = s_ref[...]
        ssum_ref[...] += jnp.sum(s, axis=0, keepdims=True)
        ssq_ref[...] += jnp.sum(s * s, axis=0, keepdims=True)


def _apply(g, nbr_t, p, we, scale, shift, m, n, tile):
    nb = n // tile
    return pl.pallas_call(
        functools.partial(_apply_body, m=m),
        grid=(nb, m),
        in_specs=[
            pl.BlockSpec((tile, A), lambda i, j: (j * nb + i, 0)),
            pl.BlockSpec((tile, NBR), lambda i, j: (j * nb + i, 0)),
            pl.BlockSpec((tile, 2 * A), lambda i, j: (i, 0)),
            pl.BlockSpec((NBR, 2 * A), lambda i, j: (0, 0)),
            pl.BlockSpec((1, 2 * A), lambda i, j: (0, 0)),
            pl.BlockSpec((1, 2 * A), lambda i, j: (0, 0)),
        ],
        out_specs=[
            pl.BlockSpec((tile, A), lambda i, j: (i, 0)),
            pl.BlockSpec((1, A), lambda i, j: (0, 0)),
            pl.BlockSpec((1, A), lambda i, j: (0, 0)),
        ],
        out_shape=[
            jax.ShapeDtypeStruct((n, A), jnp.float32),
            jax.ShapeDtypeStruct((1, A), jnp.float32),
            jax.ShapeDtypeStruct((1, A), jnp.float32),
        ],
    )(g, nbr_t, p, we, scale, shift)


def _final_body(atom_ref, s_ref, sc_ref, sh_ref, out_ref):
    out_ref[...] = jnp.maximum(
        atom_ref[...] + s_ref[...] * sc_ref[...] + sh_ref[...], 0.0)


def _final(atom, s, scale2, shift2, tile):
    n = atom.shape[0]
    return pl.pallas_call(
        _final_body,
        grid=(n // tile,),
        in_specs=[
            pl.BlockSpec((tile, A), lambda i: (i, 0)),
            pl.BlockSpec((tile, A), lambda i: (i, 0)),
            pl.BlockSpec((1, A), lambda i: (0, 0)),
            pl.BlockSpec((1, A), lambda i: (0, 0)),
        ],
        out_specs=pl.BlockSpec((tile, A), lambda i: (i, 0)),
        out_shape=jax.ShapeDtypeStruct((n, A), jnp.float32),
    )(atom, s, scale2, shift2)


def kernel(atom_in_fea, nbr_fea, nbr_fea_idx, W_full, b_full,
           bn1_gamma, bn1_beta, bn2_gamma, bn2_beta):
    n, m = nbr_fea_idx.shape
    ws = W_full[:A]
    wn = W_full[A:2 * A]
    we = W_full[2 * A:]
    b2d = b_full.reshape(1, 2 * A)

    p, t = _projections(atom_in_fea, ws, wn, b2d, tile=2000)

    idx_t = nbr_fea_idx.astype(jnp.int32).T.reshape(-1)
    nbr_t = nbr_fea.transpose(1, 0, 2).reshape(m * n, NBR)

    g = _make_sc_gather(m * n, A, chunk=200, dtype=jnp.int32)(t, idx_t)

    vsum, vsq = _stats(g, nbr_t, p, we, m, n, tile=2000)
    cnt = float(n * m)
    mu = vsum / cnt
    var = vsq / cnt - mu * mu
    scale = (bn1_gamma / jnp.sqrt(var + EPS)).reshape(1, 2 * A)
    shift = (bn1_beta - mu * scale).reshape(1, 2 * A)

    s, ssum, ssq = _apply(g, nbr_t, p, we, scale, shift, m, n, tile=2000)
    mu2 = ssum / float(n)
    var2 = ssq / float(n) - mu2 * mu2
    scale2 = (bn2_gamma / jnp.sqrt(var2 + EPS)).reshape(1, A)
    shift2 = (bn2_beta - mu2 * scale2).reshape(1, A)

    return _final(atom_in_fea, s, scale2, shift2, tile=2000)


# speedup vs baseline: 2.6059x; 1.0853x over previous
import functools
import jax
import jax.numpy as jnp
from jax import lax
from jax.experimental import pallas as pl
from jax.experimental.pallas import tpu as pltpu
from jax.experimental.pallas import tpu_sc as plsc

A = 128
NBR = 16
EPS = 1e-5


_MASK_HI = -65536


def _unpack(word):
    lo = lax.bitcast_convert_type(jnp.left_shift(word, 16), jnp.float32)
    hi = lax.bitcast_convert_type(jnp.bitwise_and(word, _MASK_HI), jnp.float32)
    return lo, hi


def _proj_body(atom_ref, ws_ref, wn_ref, b_ref, p_ref, t_ref):
    x = atom_ref[...]
    p_ref[...] = jnp.dot(x, ws_ref[...], preferred_element_type=jnp.float32) + b_ref[...]
    t = jnp.dot(x, wn_ref[...], preferred_element_type=jnp.float32)
    lo = lax.bitcast_convert_type(t[:, :A], jnp.int32) + 0x8000
    hi = lax.bitcast_convert_type(t[:, A:], jnp.int32) + 0x8000
    t_ref[...] = jnp.bitwise_or(
        jnp.bitwise_and(hi, _MASK_HI), lax.shift_right_logical(lo, 16))


def _projections(atom, ws, wn, b2d, tile):
    n = atom.shape[0]
    grid = (n // tile,)
    return pl.pallas_call(
        _proj_body,
        grid=grid,
        in_specs=[
            pl.BlockSpec((tile, A), lambda i: (i, 0)),
            pl.BlockSpec((A, 2 * A), lambda i: (0, 0)),
            pl.BlockSpec((A, 2 * A), lambda i: (0, 0)),
            pl.BlockSpec((1, 2 * A), lambda i: (0, 0)),
        ],
        out_specs=[
            pl.BlockSpec((tile, 2 * A), lambda i: (i, 0)),
            pl.BlockSpec((tile, A), lambda i: (i, 0)),
        ],
        out_shape=[
            jax.ShapeDtypeStruct((n, 2 * A), jnp.float32),
            jax.ShapeDtypeStruct((n, A), jnp.int32),
        ],
    )(atom, ws, wn, b2d)


def _make_sc_gather(nrows, d, chunk, dtype):
    info = plsc.get_sparse_core_info()
    nw = info.num_cores * info.num_subcores
    per_w = nrows // nw
    nch = per_w // chunk
    mesh = plsc.VectorSubcoreMesh(core_axis_name="c", subcore_axis_name="s")

    @functools.partial(
        pl.kernel,
        mesh=mesh,
        out_type=jax.ShapeDtypeStruct((nrows, d), dtype),
        scratch_types=[
            pltpu.VMEM((per_w,), jnp.int32),
            pltpu.VMEM((chunk, d), dtype),
            pltpu.VMEM((chunk, d), dtype),
            pltpu.SemaphoreType.DMA,
            pltpu.SemaphoreType.DMA,
            pltpu.SemaphoreType.DMA,
            pltpu.SemaphoreType.DMA,
        ],
    )
    def gk(t_hbm, idx_hbm, out_hbm, idx_all, r0, r1, sg0, sg1, sw0, sw1):
        wid = lax.axis_index("s") * info.num_cores + lax.axis_index("c")
        base = wid * per_w
        pltpu.sync_copy(idx_hbm.at[pl.ds(base, per_w)], idx_all)
        bufs = (r0, r1)
        sgs = (sg0, sg1)
        sws = (sw0, sw1)

        def start_gather(k, b):
            pltpu.async_copy(
                t_hbm.at[idx_all.at[pl.ds(k * chunk, chunk)]], bufs[b], sgs[b])

        for b in range(2):
            start_gather(b, b)

        def body(o, carry):
            for b in range(2):
                k = o * 2 + b
                pltpu.make_async_copy(
                    t_hbm.at[idx_all.at[pl.ds(k * chunk, chunk)]],
                    bufs[b], sgs[b]).wait()
                dst = out_hbm.at[pl.ds(base + k * chunk, chunk)]
                pltpu.async_copy(bufs[b], dst, sws[b])
                pltpu.make_async_copy(bufs[b], dst, sws[b]).wait()

                @pl.when(k + 2 < nch)
                def _():
                    start_gather(k + 2, b)
            return carry

        lax.fori_loop(0, nch // 2, body, 0)

    return gk


def _stats_body(g_ref, nbr_ref, p_ref, we_ref, sum_ref, sq_ref):
    n = pl.program_id(0)
    j = pl.program_id(1)
    tf, tc = _unpack(g_ref[...])
    ep = p_ref[...] + jnp.dot(
        nbr_ref[...], we_ref[...], preferred_element_type=jnp.float32)
    v = jnp.concatenate([tf + ep[:, :A], tc + ep[:, A:]], axis=1)

    @pl.when(jnp.logical_and(j == 0, n == 0))
    def _():
        sum_ref[...] = jnp.zeros_like(sum_ref)
        sq_ref[...] = jnp.zeros_like(sq_ref)

    sum_ref[...] += jnp.sum(v, axis=0, keepdims=True)
    sq_ref[...] += jnp.sum(v * v, axis=0, keepdims=True)


def _stats(g, nbr_t, p, we, m, n, tile):
    nb = n // tile
    return pl.pallas_call(
        _stats_body,
        grid=(nb, m),
        in_specs=[
            pl.BlockSpec((tile, A), lambda i, j: (j * nb + i, 0)),
            pl.BlockSpec((tile, NBR), lambda i, j: (j * nb + i, 0)),
            pl.BlockSpec((tile, 2 * A), lambda i, j: (i, 0)),
            pl.BlockSpec((NBR, 2 * A), lambda i, j: (0, 0)),
        ],
        out_specs=[
            pl.BlockSpec((1, 2 * A), lambda i, j: (0, 0)),
            pl.BlockSpec((1, 2 * A), lambda i, j: (0, 0)),
        ],
        out_shape=[
            jax.ShapeDtypeStruct((1, 2 * A), jnp.float32),
            jax.ShapeDtypeStruct((1, 2 * A), jnp.float32),
        ],
    )(g, nbr_t, p, we)


def _apply_body(g_ref, nbr_ref, p_ref, we_ref, sc_ref, sh_ref,
                s_ref, ssum_ref, ssq_ref, *, m):
    n = pl.program_id(0)
    j = pl.program_id(1)
    tf, tc = _unpack(g_ref[...])
    ep = p_ref[...] + jnp.dot(
        nbr_ref[...], we_ref[...], preferred_element_type=jnp.float32)
    sc = sc_ref[...]
    sh = sh_ref[...]
    uf = (tf + ep[:, :A]) * sc[:, :A] + sh[:, :A]
    uc = (tc + ep[:, A:]) * sc[:, A:] + sh[:, A:]
    prod = jax.nn.sigmoid(uf) * jnp.maximum(uc, 0.0)

    @pl.when(j == 0)
    def _():
        s_ref[...] = prod

    @pl.when(j > 0)
    def _():
        s_ref[...] += prod

    @pl.when(jnp.logical_and(n == 0, j == m - 1))
    def _():
        ssum_ref[...] = jnp.zeros_like(ssum_ref)
        ssq_ref[...] = jnp.zeros_like(ssq_ref)

    @pl.when(j == m - 1)
    def _():
        s = s_ref[...]
        ssum_ref[...] += jnp.sum(s, axis=0, keepdims=True)
        ssq_ref[...] += jnp.sum(s * s, axis=0, keepdims=True)


def _apply(g, nbr_t, p, we, scale, shift, m, n, tile):
    nb = n // tile
    return pl.pallas_call(
        functools.partial(_apply_body, m=m),
        grid=(nb, m),
        in_specs=[
            pl.BlockSpec((tile, A), lambda i, j: (j * nb + i, 0)),
            pl.BlockSpec((tile, NBR), lambda i, j: (j * nb + i, 0)),
            pl.BlockSpec((tile, 2 * A), lambda i, j: (i, 0)),
            pl.BlockSpec((NBR, 2 * A), lambda i, j: (0, 0)),
            pl.BlockSpec((1, 2 * A), lambda i, j: (0, 0)),
            pl.BlockSpec((1, 2 * A), lambda i, j: (0, 0)),
        ],
        out_specs=[
            pl.BlockSpec((tile, A), lambda i, j: (i, 0)),
            pl.BlockSpec((1, A), lambda i, j: (0, 0)),
            pl.BlockSpec((1, A), lambda i, j: (0, 0)),
        ],
        out_shape=[
            jax.ShapeDtypeStruct((n, A), jnp.float32),
            jax.ShapeDtypeStruct((1, A), jnp.float32),
            jax.ShapeDtypeStruct((1, A), jnp.float32),
        ],
    )(g, nbr_t, p, we, scale, shift)


def _final_body(atom_ref, s_ref, sc_ref, sh_ref, out_ref):
    out_ref[...] = jnp.maximum(
        atom_ref[...] + s_ref[...] * sc_ref[...] + sh_ref[...], 0.0)


def _final(atom, s, scale2, shift2, tile):
    n = atom.shape[0]
    return pl.pallas_call(
        _final_body,
        grid=(n // tile,),
        in_specs=[
            pl.BlockSpec((tile, A), lambda i: (i, 0)),
            pl.BlockSpec((tile, A), lambda i: (i, 0)),
            pl.BlockSpec((1, A), lambda i: (0, 0)),
            pl.BlockSpec((1, A), lambda i: (0, 0)),
        ],
        out_specs=pl.BlockSpec((tile, A), lambda i: (i, 0)),
        out_shape=jax.ShapeDtypeStruct((n, A), jnp.float32),
    )(atom, s, scale2, shift2)


def kernel(atom_in_fea, nbr_fea, nbr_fea_idx, W_full, b_full,
           bn1_gamma, bn1_beta, bn2_gamma, bn2_beta):
    n, m = nbr_fea_idx.shape
    ws = W_full[:A]
    wn = W_full[A:2 * A]
    we = W_full[2 * A:]
    b2d = b_full.reshape(1, 2 * A)

    p, t = _projections(atom_in_fea, ws, wn, b2d, tile=2000)

    idx_t = nbr_fea_idx.astype(jnp.int32).T.reshape(-1)
    nbr_t = nbr_fea.transpose(1, 0, 2).reshape(m * n, NBR)

    g = _make_sc_gather(m * n, A, chunk=200, dtype=jnp.int32)(t, idx_t)

    vsum, vsq = _stats(g, nbr_t, p, we, m, n, tile=2000)
    cnt = float(n * m)
    mu = vsum / cnt
    var = vsq / cnt - mu * mu
    scale = (bn1_gamma / jnp.sqrt(var + EPS)).reshape(1, 2 * A)
    shift = (bn1_beta - mu * scale).reshape(1, 2 * A)

    s, ssum, ssq = _apply(g, nbr_t, p, we, scale, shift, m, n, tile=2000)
    mu2 = ssum / float(n)
    var2 = ssq / float(n) - mu2 * mu2
    scale2 = (bn2_gamma / jnp.sqrt(var2 + EPS)).reshape(1, A)
    shift2 = (bn2_beta - mu2 * scale2).reshape(1, A)

    return _final(atom_in_fea, s, scale2, shift2, tile=2000)


# speedup vs baseline: 3.2910x; 1.2629x over previous
import functools
import jax
import jax.numpy as jnp
from jax import lax
from jax.experimental import pallas as pl
from jax.experimental.pallas import tpu as pltpu
from jax.experimental.pallas import tpu_sc as plsc

A = 128
NBR = 16
EPS = 1e-5


_MASK_HI = -65536


def _unpack(word):
    lo = lax.bitcast_convert_type(jnp.left_shift(word, 16), jnp.float32)
    hi = lax.bitcast_convert_type(jnp.bitwise_and(word, _MASK_HI), jnp.float32)
    return lo, hi


def _proj_body(atom_ref, ws_ref, wn_ref, b_ref, p_ref, t_ref):
    x = atom_ref[...]
    p_ref[...] = jnp.dot(x, ws_ref[...], preferred_element_type=jnp.float32) + b_ref[...]
    t = jnp.dot(x, wn_ref[...], preferred_element_type=jnp.float32)
    lo = lax.bitcast_convert_type(t[:, :A], jnp.int32) + 0x8000
    hi = lax.bitcast_convert_type(t[:, A:], jnp.int32) + 0x8000
    t_ref[...] = jnp.bitwise_or(
        jnp.bitwise_and(hi, _MASK_HI), lax.shift_right_logical(lo, 16))


def _projections(atom, ws, wn, b2d, tile):
    n = atom.shape[0]
    grid = (n // tile,)
    return pl.pallas_call(
        _proj_body,
        grid=grid,
        in_specs=[
            pl.BlockSpec((tile, A), lambda i: (i, 0)),
            pl.BlockSpec((A, 2 * A), lambda i: (0, 0)),
            pl.BlockSpec((A, 2 * A), lambda i: (0, 0)),
            pl.BlockSpec((1, 2 * A), lambda i: (0, 0)),
        ],
        out_specs=[
            pl.BlockSpec((tile, 2 * A), lambda i: (i, 0)),
            pl.BlockSpec((tile, A), lambda i: (i, 0)),
        ],
        out_shape=[
            jax.ShapeDtypeStruct((n, 2 * A), jnp.float32),
            jax.ShapeDtypeStruct((n, A), jnp.int32),
        ],
    )(atom, ws, wn, b2d)


def _make_sc_gather(nrows, d, chunk, dtype):
    info = plsc.get_sparse_core_info()
    nw = info.num_cores * info.num_subcores
    per_w = nrows // nw
    nch = per_w // chunk
    mesh = plsc.VectorSubcoreMesh(core_axis_name="c", subcore_axis_name="s")

    @functools.partial(
        pl.kernel,
        mesh=mesh,
        out_type=jax.ShapeDtypeStruct((nrows, d), dtype),
        scratch_types=[
            pltpu.VMEM((per_w,), jnp.int32),
            pltpu.VMEM((chunk, d), dtype),
            pltpu.VMEM((chunk, d), dtype),
            pltpu.SemaphoreType.DMA,
            pltpu.SemaphoreType.DMA,
            pltpu.SemaphoreType.DMA,
            pltpu.SemaphoreType.DMA,
        ],
    )
    def gk(t_hbm, idx_hbm, out_hbm, idx_all, r0, r1, sg0, sg1, sw0, sw1):
        wid = lax.axis_index("s") * info.num_cores + lax.axis_index("c")
        base = wid * per_w
        pltpu.sync_copy(idx_hbm.at[pl.ds(base, per_w)], idx_all)
        bufs = (r0, r1)
        sgs = (sg0, sg1)
        sws = (sw0, sw1)

        def start_gather(k, b):
            pltpu.async_copy(
                t_hbm.at[idx_all.at[pl.ds(k * chunk, chunk)]], bufs[b], sgs[b])

        for b in range(2):
            start_gather(b, b)

        def body(o, carry):
            for b in range(2):
                k = o * 2 + b
                pltpu.make_async_copy(
                    t_hbm.at[idx_all.at[pl.ds(k * chunk, chunk)]],
                    bufs[b], sgs[b]).wait()
                dst = out_hbm.at[pl.ds(base + k * chunk, chunk)]
                pltpu.async_copy(bufs[b], dst, sws[b])
                pltpu.make_async_copy(bufs[b], dst, sws[b]).wait()

                @pl.when(k + 2 < nch)
                def _():
                    start_gather(k + 2, b)
            return carry

        lax.fori_loop(0, nch // 2, body, 0)

    return gk


def _edge_halves(g_ref, nbr_ref, p_ref, we_ref):
    tn, m_, a = g_ref.shape
    tf, tc = _unpack(g_ref[...].reshape(tn * m_, a))
    ep = jnp.dot(nbr_ref[...].reshape(tn * m_, NBR), we_ref[...],
                 preferred_element_type=jnp.float32)
    p = p_ref[...]
    pf = jnp.broadcast_to(p[:, None, :A], (tn, m_, a)).reshape(tn * m_, a)
    pc = jnp.broadcast_to(p[:, None, A:], (tn, m_, a)).reshape(tn * m_, a)
    return tf + pf + ep[:, :A], tc + pc + ep[:, A:]


def _stats_body(g_ref, nbr_ref, p_ref, we_ref, sum_ref, sq_ref):
    vf, vc = _edge_halves(g_ref, nbr_ref, p_ref, we_ref)

    @pl.when(pl.program_id(0) == 0)
    def _():
        sum_ref[...] = jnp.zeros_like(sum_ref)
        sq_ref[...] = jnp.zeros_like(sq_ref)

    sum_ref[...] += jnp.concatenate(
        [jnp.sum(vf, axis=0, keepdims=True),
         jnp.sum(vc, axis=0, keepdims=True)], axis=1)
    sq_ref[...] += jnp.concatenate(
        [jnp.sum(vf * vf, axis=0, keepdims=True),
         jnp.sum(vc * vc, axis=0, keepdims=True)], axis=1)


def _stats(g3, nbr, p, we, m, n, tile):
    return pl.pallas_call(
        _stats_body,
        grid=(n // tile,),
        in_specs=[
            pl.BlockSpec((tile, m, A), lambda i: (i, 0, 0)),
            pl.BlockSpec((tile, m, NBR), lambda i: (i, 0, 0)),
            pl.BlockSpec((tile, 2 * A), lambda i: (i, 0)),
            pl.BlockSpec((NBR, 2 * A), lambda i: (0, 0)),
        ],
        out_specs=[
            pl.BlockSpec((1, 2 * A), lambda i: (0, 0)),
            pl.BlockSpec((1, 2 * A), lambda i: (0, 0)),
        ],
        out_shape=[
            jax.ShapeDtypeStruct((1, 2 * A), jnp.float32),
            jax.ShapeDtypeStruct((1, 2 * A), jnp.float32),
        ],
    )(g3, nbr, p, we)


def _apply_body(g_ref, nbr_ref, p_ref, we_ref, sc_ref, sh_ref,
                s_ref, ssum_ref, ssq_ref):
    tn, m_, a = g_ref.shape
    vf, vc = _edge_halves(g_ref, nbr_ref, p_ref, we_ref)
    sc = sc_ref[...]
    sh = sh_ref[...]
    uf = vf * sc[:, :A] + sh[:, :A]
    uc = vc * sc[:, A:] + sh[:, A:]
    prod = jax.nn.sigmoid(uf) * jnp.maximum(uc, 0.0)
    s = jnp.sum(prod.reshape(tn, m_, a), axis=1)
    s_ref[...] = s

    @pl.when(pl.program_id(0) == 0)
    def _():
        ssum_ref[...] = jnp.zeros_like(ssum_ref)
        ssq_ref[...] = jnp.zeros_like(ssq_ref)

    ssum_ref[...] += jnp.sum(s, axis=0, keepdims=True)
    ssq_ref[...] += jnp.sum(s * s, axis=0, keepdims=True)


def _apply(g3, nbr, p, we, scale, shift, m, n, tile):
    return pl.pallas_call(
        _apply_body,
        grid=(n // tile,),
        in_specs=[
            pl.BlockSpec((tile, m, A), lambda i: (i, 0, 0)),
            pl.BlockSpec((tile, m, NBR), lambda i: (i, 0, 0)),
            pl.BlockSpec((tile, 2 * A), lambda i: (i, 0)),
            pl.BlockSpec((NBR, 2 * A), lambda i: (0, 0)),
            pl.BlockSpec((1, 2 * A), lambda i: (0, 0)),
            pl.BlockSpec((1, 2 * A), lambda i: (0, 0)),
        ],
        out_specs=[
            pl.BlockSpec((tile, A), lambda i: (i, 0)),
            pl.BlockSpec((1, A), lambda i: (0, 0)),
            pl.BlockSpec((1, A), lambda i: (0, 0)),
        ],
        out_shape=[
            jax.ShapeDtypeStruct((n, A), jnp.float32),
            jax.ShapeDtypeStruct((1, A), jnp.float32),
            jax.ShapeDtypeStruct((1, A), jnp.float32),
        ],
    )(g3, nbr, p, we, scale, shift)


def _final_body(atom_ref, s_ref, sc_ref, sh_ref, out_ref):
    out_ref[...] = jnp.maximum(
        atom_ref[...] + s_ref[...] * sc_ref[...] + sh_ref[...], 0.0)


def _final(atom, s, scale2, shift2, tile):
    n = atom.shape[0]
    return pl.pallas_call(
        _final_body,
        grid=(n // tile,),
        in_specs=[
            pl.BlockSpec((tile, A), lambda i: (i, 0)),
            pl.BlockSpec((tile, A), lambda i: (i, 0)),
            pl.BlockSpec((1, A), lambda i: (0, 0)),
            pl.BlockSpec((1, A), lambda i: (0, 0)),
        ],
        out_specs=pl.BlockSpec((tile, A), lambda i: (i, 0)),
        out_shape=jax.ShapeDtypeStruct((n, A), jnp.float32),
    )(atom, s, scale2, shift2)


def kernel(atom_in_fea, nbr_fea, nbr_fea_idx, W_full, b_full,
           bn1_gamma, bn1_beta, bn2_gamma, bn2_beta):
    n, m = nbr_fea_idx.shape
    ws = W_full[:A]
    wn = W_full[A:2 * A]
    we = W_full[2 * A:]
    b2d = b_full.reshape(1, 2 * A)

    p, t = _projections(atom_in_fea, ws, wn, b2d, tile=2000)

    idx_flat = nbr_fea_idx.astype(jnp.int32).reshape(-1)

    g = _make_sc_gather(m * n, A, chunk=200, dtype=jnp.int32)(t, idx_flat)
    g3 = g.reshape(n, m, A)

    vsum, vsq = _stats(g3, nbr_fea, p, we, m, n, tile=400)
    cnt = float(n * m)
    mu = vsum / cnt
    var = vsq / cnt - mu * mu
    scale = (bn1_gamma / jnp.sqrt(var + EPS)).reshape(1, 2 * A)
    shift = (bn1_beta - mu * scale).reshape(1, 2 * A)

    s, ssum, ssq = _apply(g3, nbr_fea, p, we, scale, shift, m, n, tile=400)
    mu2 = ssum / float(n)
    var2 = ssq / float(n) - mu2 * mu2
    scale2 = (bn2_gamma / jnp.sqrt(var2 + EPS)).reshape(1, A)
    shift2 = (bn2_beta - mu2 * scale2).reshape(1, A)

    return _final(atom_in_fea, s, scale2, shift2, tile=2000)
